# baseline (speedup 1.0000x reference)
import functools

import jax
import jax.numpy as jnp
from jax import lax
from jax.experimental import pallas as pl
from jax.experimental.pallas import tpu as pltpu
from jax.experimental.pallas import tpu_sc as plsc

_BATCH = 16384
_D = 64
_LANES = 16
_NC = 2
_NS = 16
_NW = _NC * _NS
_BPW = _BATCH // _NW
_CH = 128
_NCH = _BPW // _CH
_SCALE = float((1.0 + 1e-3) ** -1.5)

_mesh = plsc.VectorSubcoreMesh(core_axis_name="c", subcore_axis_name="s")


@functools.partial(
    pl.kernel,
    out_type=jax.ShapeDtypeStruct((_BATCH,), jnp.float32),
    mesh=_mesh,
    compiler_params=pltpu.CompilerParams(
        needs_layout_passes=False, use_tc_tiling_on_sc=False),
    scratch_types=[
        pltpu.VMEM((_NCH, _CH), jnp.int32),
        pltpu.VMEM((_NCH, _CH), jnp.int32),
        pltpu.VMEM((_NCH, _CH), jnp.int32),
        pltpu.VMEM((_BPW, _D), jnp.float32),
        pltpu.VMEM((_BPW, _D), jnp.float32),
        pltpu.VMEM((_BPW, _D), jnp.float32),
        pltpu.VMEM((_BPW,), jnp.float32),
        pltpu.SemaphoreType.DMA,
    ],
)
def _sc_score(s_idx_hbm, p_idx_hbm, o_idx_hbm, ent_hbm, rel_hbm, out_hbm,
              si_v, pi_v, oi_v, s_v, p_v, o_v, out_v, sem):
    wid = lax.axis_index("s") * _NC + lax.axis_index("c")
    base = wid * _BPW

    pltpu.sync_copy(s_idx_hbm.at[wid], si_v)
    pltpu.sync_copy(p_idx_hbm.at[wid], pi_v)
    pltpu.sync_copy(o_idx_hbm.at[wid], oi_v)

    copies = []
    for j in range(_NCH):
        rows = pl.ds(j * _CH, _CH)
        copies.append(pltpu.async_copy(ent_hbm.at[si_v.at[j]], s_v.at[rows], sem))
        copies.append(pltpu.async_copy(rel_hbm.at[pi_v.at[j]], p_v.at[rows], sem))
        copies.append(pltpu.async_copy(ent_hbm.at[oi_v.at[j]], o_v.at[rows], sem))
    for c in copies:
        c.wait()

    lanes = lax.iota(jnp.int32, _LANES)

    def group_body(g, _):
        row0 = g * _LANES

        def row_body(j, vec):
            i = row0 + j
            acc = None
            for c in range(_D // _LANES):
                d = pl.ds(c * _LANES, _LANES)
                t = s_v[i, d] * p_v[i, d] * o_v[i, d]
                acc = t if acc is None else acc + t
            return jnp.where(lanes == j, jnp.sum(acc), vec)

        vec = lax.fori_loop(0, _LANES, row_body, jnp.zeros((_LANES,), jnp.float32))
        out_v[pl.ds(row0, _LANES)] = vec * _SCALE
        return 0

    lax.fori_loop(0, _BPW // _LANES, group_body, 0)

    pltpu.sync_copy(out_v, out_hbm.at[pl.ds(base, _BPW)])


def kernel(inputs, entity_table, relation_table):
    idx = inputs.astype(jnp.int32)
    s_idx = idx[:, 0].reshape(_NW, _NCH, _CH)
    p_idx = idx[:, 1].reshape(_NW, _NCH, _CH)
    o_idx = idx[:, 2].reshape(_NW, _NCH, _CH)
    return _sc_score(s_idx, p_idx, o_idx, entity_table, relation_table)
